# Initial kernel scaffold; baseline (speedup 1.0000x reference)
import jax
import jax.numpy as jnp
from jax.experimental import pallas as pl


def kernel(img, rand_boxes_init, pseudo_scores, num_boxes_per_img):
    raise NotImplementedError("write your pallas kernel here")


# speedup vs baseline: 1265.2172x; 1265.2172x over previous
import functools

import jax
import jax.numpy as jnp
from jax import lax
from jax.experimental import pallas as pl
from jax.experimental.pallas import tpu as pltpu

_NMS_THR = 0.7
_NUM_INIT = 5000
_PAD = 5120
_ROWS = 8
_COLS = _PAD // _ROWS
_MAX_FINAL = 50
_MAX_ITERS = 49
_NEG = -1.0e30
_IMG_HW = 512.0
_MIN_SIDE = _IMG_HW * 0.1


def _nms_body(npb_ref, r_ref, s_ref, out_ref, cnt_ref):
    r0 = r_ref[0, 0]
    r1 = r_ref[0, 1]
    r2 = r_ref[0, 2]
    r3 = r_ref[0, 3]
    x1 = jnp.minimum(r0, r2) * _IMG_HW
    x2 = jnp.maximum(r0, r2) * _IMG_HW
    y1 = jnp.minimum(r1, r3) * _IMG_HW
    y2 = jnp.maximum(r1, r3) * _IMG_HW
    w = x2 - x1
    h = y2 - y1
    valid = (h > _MIN_SIDE) & (w > _MIN_SIDE)
    area = w * h
    s0 = jnp.where(valid, s_ref[0], _NEG)

    lin = (lax.broadcasted_iota(jnp.int32, (_ROWS, _COLS), 0) * _COLS
           + lax.broadcasted_iota(jnp.int32, (_ROWS, _COLS), 1))
    lin_out = (lax.broadcasted_iota(jnp.int32, (_ROWS, 64), 0) * 64
               + lax.broadcasted_iota(jnp.int32, (_ROWS, 64), 1))
    big = jnp.int32(1 << 30)

    def step(t, carry):
        s, cnt, o0, o1, o2, o3 = carry
        m = jnp.max(s)
        alive = m > (_NEG * 0.5)
        am = jnp.min(jnp.where(s == m, lin, big))
        sel = lin == am
        bx1 = jnp.sum(jnp.where(sel, x1, 0.0))
        by1 = jnp.sum(jnp.where(sel, y1, 0.0))
        bx2 = jnp.sum(jnp.where(sel, x2, 0.0))
        by2 = jnp.sum(jnp.where(sel, y2, 0.0))
        barea = jnp.sum(jnp.where(sel, area, 0.0))
        xx1 = jnp.maximum(x1, bx1)
        yy1 = jnp.maximum(y1, by1)
        xx2 = jnp.minimum(x2, bx2)
        yy2 = jnp.minimum(y2, by2)
        inter = jnp.clip(xx2 - xx1, 0.0) * jnp.clip(yy2 - yy1, 0.0)
        iou = inter / (area + barea - inter + 1e-9)
        supp = (iou > _NMS_THR) & alive
        s = jnp.where(supp, _NEG, s)
        oh = (lin_out == t) & alive
        o0 = jnp.where(oh, bx1, o0)
        o1 = jnp.where(oh, by1, o1)
        o2 = jnp.where(oh, bx2, o2)
        o3 = jnp.where(oh, by2, o3)
        cnt = cnt + alive.astype(jnp.int32)
        return (s, cnt, o0, o1, o2, o3)

    zeros = jnp.zeros((_ROWS, 64), jnp.float32)
    s, cnt, o0, o1, o2, o3 = lax.fori_loop(
        0, _MAX_ITERS, step, (s0, jnp.int32(0), zeros, zeros, zeros, zeros))

    n_final = jnp.minimum(cnt, npb_ref[0, 0, 0])
    keep_row = lin_out < n_final
    out_ref[0, 0] = jnp.where(keep_row, o0, 0.0)
    out_ref[0, 1] = jnp.where(keep_row, o1, 0.0)
    out_ref[0, 2] = jnp.where(keep_row, o2, 0.0)
    out_ref[0, 3] = jnp.where(keep_row, o3, 0.0)
    cnt_ref[0, 0, 0] = n_final


def _split_body(a_ref, b_ref, o1_ref, o2_ref):
    o1_ref[...] = a_ref[...]
    o2_ref[...] = b_ref[...]


@functools.partial(jax.jit, static_argnames=())
def kernel(img, rand_boxes_init, pseudo_scores, num_boxes_per_img):
    nimg = img.shape[0]
    n = rand_boxes_init.shape[1]

    rt = jnp.moveaxis(rand_boxes_init, 2, 1)
    rt = jnp.pad(rt, ((0, 0), (0, 0), (0, _PAD - n)))
    r4 = rt.reshape(nimg, 4, _ROWS, _COLS)
    sc = jnp.pad(pseudo_scores, ((0, 0), (0, _PAD - n))).reshape(
        nimg, _ROWS, _COLS)
    npb = num_boxes_per_img.reshape(nimg, 1, 1)

    planes, cnts = pl.pallas_call(
        _nms_body,
        grid=(nimg,),
        in_specs=[
            pl.BlockSpec((1, 1, 1), lambda i: (i, 0, 0),
                         memory_space=pltpu.SMEM),
            pl.BlockSpec((1, 4, _ROWS, _COLS), lambda i: (i, 0, 0, 0)),
            pl.BlockSpec((1, _ROWS, _COLS), lambda i: (i, 0, 0)),
        ],
        out_specs=[
            pl.BlockSpec((1, 4, _ROWS, 64), lambda i: (i, 0, 0, 0)),
            pl.BlockSpec((1, 1, 1), lambda i: (i, 0, 0),
                         memory_space=pltpu.SMEM),
        ],
        out_shape=[
            jax.ShapeDtypeStruct((nimg, 4, _ROWS, 64), jnp.float32),
            jax.ShapeDtypeStruct((nimg, 1, 1), jnp.int32),
        ],
    )(npb, r4, sc)

    boxes = planes.reshape(nimg, 4, _ROWS * 64)[:, :, :_MAX_FINAL]
    rand_box_1 = jnp.moveaxis(boxes, 1, 2)
    counts = cnts[:, 0, 0]

    ch = img.shape[1] // 2
    hw = img.shape[2:]
    img_1, img_2 = pl.pallas_call(
        _split_body,
        grid=(nimg, ch),
        in_specs=[
            pl.BlockSpec((1, 1) + hw, lambda i, c: (i, c, 0, 0)),
            pl.BlockSpec((1, 1) + hw, lambda i, c, _ch=ch: (i, c + _ch, 0, 0)),
        ],
        out_specs=[
            pl.BlockSpec((1, 1) + hw, lambda i, c: (i, c, 0, 0)),
            pl.BlockSpec((1, 1) + hw, lambda i, c: (i, c, 0, 0)),
        ],
        out_shape=[
            jax.ShapeDtypeStruct((nimg, ch) + hw, img.dtype),
            jax.ShapeDtypeStruct((nimg, ch) + hw, img.dtype),
        ],
    )(img, img)

    return (rand_box_1, rand_box_1, img_1, img_2, counts)


# speedup vs baseline: 1805.7331x; 1.4272x over previous
import functools

import jax
import jax.numpy as jnp
from jax import lax
from jax.experimental import pallas as pl
from jax.experimental.pallas import tpu as pltpu
from jax.experimental.pallas import tpu_sc as plsc

_NMS_THR = 0.7
_NUM_INIT = 5000
_PAD = 5120
_MAX_FINAL = 50
_ROUNDS = 49
_NEG = -1.0e30
_ALIVE_THR = -1.0e29
_IMG_HW = 512.0
_MIN_SIDE = _IMG_HW * 0.1

_NTILE = 8
_CHUNK = _PAD // _NTILE
_NCH = _CHUNK // 16


def _sc_nms_body(pkg_hbm, npb_hbm, out_hbm,
                 pkg_v, geo_v, cand_v, all8_v, out_v, npb_v, shared):
    sid = lax.axis_index("s")
    cid = lax.axis_index("c")
    img = cid * 2 + sid // _NTILE
    g = sid % _NTILE
    base = (sid // _NTILE) * _NTILE

    pltpu.sync_copy(pkg_hbm.at[img * _NTILE + g], pkg_v)
    pltpu.sync_copy(npb_hbm, npb_v)

    iota16 = lax.iota(jnp.int32, 16)
    npb = jnp.max(
        jnp.where(iota16 == img, npb_v[...].astype(jnp.float32), 0.0)
    ).astype(jnp.int32)

    def geo(k, _):
        sl = pl.ds(k * 16, 16)
        r0 = pkg_v[pl.ds(k * 16, 16)]
        r1 = pkg_v[pl.ds(_CHUNK + k * 16, 16)]
        r2 = pkg_v[pl.ds(2 * _CHUNK + k * 16, 16)]
        r3 = pkg_v[pl.ds(3 * _CHUNK + k * 16, 16)]
        s = pkg_v[pl.ds(4 * _CHUNK + k * 16, 16)]
        x1 = jnp.minimum(r0, r2) * _IMG_HW
        x2 = jnp.maximum(r0, r2) * _IMG_HW
        y1 = jnp.minimum(r1, r3) * _IMG_HW
        y2 = jnp.maximum(r1, r3) * _IMG_HW
        w = x2 - x1
        h = y2 - y1
        valid = (h > _MIN_SIDE) & (w > _MIN_SIDE)
        geo_v[pl.ds(k * 16, 16)] = x1
        geo_v[pl.ds(_CHUNK + k * 16, 16)] = y1
        geo_v[pl.ds(2 * _CHUNK + k * 16, 16)] = x2
        geo_v[pl.ds(3 * _CHUNK + k * 16, 16)] = y2
        geo_v[pl.ds(4 * _CHUNK + k * 16, 16)] = w * h
        geo_v[pl.ds(5 * _CHUNK + k * 16, 16)] = jnp.where(valid, s, _NEG)
        return 0

    lax.fori_loop(0, _NCH, geo, 0)

    zeros16 = jnp.zeros((16,), jnp.float32)
    plsc.store_scatter(out_v, [_ROUNDS * 16 + iota16], zeros16)

    offs = (jnp.where(iota16 == 2, 0, 0)
            + jnp.where(iota16 == 3, _CHUNK, 0)
            + jnp.where(iota16 == 4, 2 * _CHUNK, 0)
            + jnp.where(iota16 == 5, 3 * _CHUNK, 0)
            + jnp.where(iota16 == 6, 4 * _CHUNK, 0))

    def rnd(t, carry):
        bx1, by1, bx2, by2, barea, sup_en, cnt = carry
        sup_b = sup_en > 0.5

        def chunk(k, mc):
            m_v, a_v = mc
            x1c = geo_v[pl.ds(k * 16, 16)]
            y1c = geo_v[pl.ds(_CHUNK + k * 16, 16)]
            x2c = geo_v[pl.ds(2 * _CHUNK + k * 16, 16)]
            y2c = geo_v[pl.ds(3 * _CHUNK + k * 16, 16)]
            ac = geo_v[pl.ds(4 * _CHUNK + k * 16, 16)]
            sc = geo_v[pl.ds(5 * _CHUNK + k * 16, 16)]
            xx1 = jnp.maximum(x1c, bx1)
            yy1 = jnp.maximum(y1c, by1)
            xx2 = jnp.minimum(x2c, bx2)
            yy2 = jnp.minimum(y2c, by2)
            inter = (jnp.maximum(xx2 - xx1, 0.0)
                     * jnp.maximum(yy2 - yy1, 0.0))
            iou = inter / (ac + barea - inter + 1e-9)
            snew = jnp.where((iou > _NMS_THR) & sup_b, _NEG, sc)
            geo_v[pl.ds(5 * _CHUNK + k * 16, 16)] = snew
            take = snew > m_v
            m_v = jnp.where(take, snew, m_v)
            a_v = jnp.where(take, (k * 16 + iota16).astype(jnp.float32), a_v)
            return m_v, a_v

        m_v, a_v = lax.fori_loop(
            0, _NCH, chunk,
            (jnp.full((16,), _NEG, jnp.float32),
             jnp.zeros((16,), jnp.float32)))

        m = jnp.max(m_v)
        am_f = jnp.min(jnp.where(m_v == m, a_v, jnp.float32(1e30)))
        am = jnp.minimum(am_f.astype(jnp.int32), _CHUNK - 1)
        gath = plsc.load_gather(geo_v, [am + offs])
        cand = jnp.where(iota16 == 0, m, gath)
        cand = jnp.where(iota16 == 1,
                         (g * _CHUNK + am).astype(jnp.float32), cand)
        cand_v[...] = cand
        pltpu.sync_copy(cand_v, shared.at[pl.ds(sid * 16, 16)])
        plsc.subcore_barrier()
        pltpu.sync_copy(shared.at[pl.ds(base * 16, _NTILE * 16)], all8_v)
        plsc.subcore_barrier()

        best_m = jnp.float32(_NEG)
        best_i = jnp.float32(1e30)
        nx1 = jnp.float32(0.0)
        ny1 = jnp.float32(0.0)
        nx2 = jnp.float32(0.0)
        ny2 = jnp.float32(0.0)
        nar = jnp.float32(0.0)
        for j in range(_NTILE):
            row = all8_v[pl.ds(16 * j, 16)]
            rm = row[0]
            ri = row[1]
            better = (rm > best_m) | ((rm == best_m) & (ri < best_i))
            best_m = jnp.where(better, rm, best_m)
            best_i = jnp.where(better, ri, best_i)
            nx1 = jnp.where(better, row[2], nx1)
            ny1 = jnp.where(better, row[3], ny1)
            nx2 = jnp.where(better, row[4], nx2)
            ny2 = jnp.where(better, row[5], ny2)
            nar = jnp.where(better, row[6], nar)

        alive = best_m > _ALIVE_THR
        alive_f = jnp.where(alive, 1.0, 0.0).astype(jnp.float32)
        write_f = jnp.where(alive & (t < npb), 1.0, 0.0).astype(jnp.float32)
        vec = jnp.where(iota16 == 0, nx1, 0.0)
        vec = jnp.where(iota16 == 1, ny1, vec)
        vec = jnp.where(iota16 == 2, nx2, vec)
        vec = jnp.where(iota16 == 3, ny2, vec)
        plsc.store_scatter(out_v, [t * 16 + iota16], vec * write_f)
        cnt = cnt + alive.astype(jnp.int32)
        return (nx1, ny1, nx2, ny2, nar, alive_f, cnt)

    z = jnp.float32(0.0)
    carry = (z, z, z, z, z, z, jnp.int32(0))
    carry = lax.fori_loop(0, _ROUNDS, rnd, carry)
    cnt = carry[6]

    n_final = jnp.minimum(cnt, npb)
    plsc.store_scatter(out_v, [_MAX_FINAL * 16 + iota16],
                       jnp.full((16,), n_final.astype(jnp.float32)))

    @pl.when(g == 0)
    def _():
        pltpu.sync_copy(out_v, out_hbm.at[img])


def _split_body(a_ref, b_ref, o1_ref, o2_ref):
    o1_ref[...] = a_ref[...]
    o2_ref[...] = b_ref[...]


@functools.partial(jax.jit, static_argnames=())
def kernel(img, rand_boxes_init, pseudo_scores, num_boxes_per_img):
    nimg = img.shape[0]
    n = rand_boxes_init.shape[1]

    rt = jnp.moveaxis(rand_boxes_init, 2, 1)
    rt = jnp.pad(rt, ((0, 0), (0, 0), (0, _PAD - n)))
    sc = jnp.pad(pseudo_scores, ((0, 0), (0, _PAD - n)))
    pkg = jnp.concatenate([rt, sc[:, None, :]], axis=1)
    pkg = pkg.reshape(nimg, 5, _NTILE, _CHUNK)
    pkg = jnp.moveaxis(pkg, 1, 2).reshape(nimg * _NTILE, 5 * _CHUNK)
    npb16 = jnp.pad(num_boxes_per_img, (0, 16 - nimg))

    mesh = plsc.VectorSubcoreMesh(core_axis_name="c", subcore_axis_name="s")
    out_flat = pl.kernel(
        _sc_nms_body,
        out_type=jax.ShapeDtypeStruct((nimg, 1024), jnp.float32),
        mesh=mesh,
        compiler_params=pltpu.CompilerParams(needs_layout_passes=False),
        scratch_types=[
            pltpu.VMEM((5 * _CHUNK,), jnp.float32),
            pltpu.VMEM((6 * _CHUNK,), jnp.float32),
            pltpu.VMEM((16,), jnp.float32),
            pltpu.VMEM((_NTILE * 16,), jnp.float32),
            pltpu.VMEM((1024,), jnp.float32),
            pltpu.VMEM((16,), jnp.int32),
            pltpu.VMEM_SHARED((256,), jnp.float32),
        ],
    )(pkg, npb16)

    rand_box_1 = out_flat[:, :_MAX_FINAL * 16].reshape(
        nimg, _MAX_FINAL, 16)[:, :, :4]
    counts = out_flat[:, _MAX_FINAL * 16].astype(jnp.int32)

    ch = img.shape[1] // 2
    hw = img.shape[2:]
    img_1, img_2 = pl.pallas_call(
        _split_body,
        grid=(nimg, ch),
        in_specs=[
            pl.BlockSpec((1, 1) + hw, lambda i, c: (i, c, 0, 0)),
            pl.BlockSpec((1, 1) + hw, lambda i, c, _ch=ch: (i, c + _ch, 0, 0)),
        ],
        out_specs=[
            pl.BlockSpec((1, 1) + hw, lambda i, c: (i, c, 0, 0)),
            pl.BlockSpec((1, 1) + hw, lambda i, c: (i, c, 0, 0)),
        ],
        out_shape=[
            jax.ShapeDtypeStruct((nimg, ch) + hw, img.dtype),
            jax.ShapeDtypeStruct((nimg, ch) + hw, img.dtype),
        ],
    )(img, img)

    return (rand_box_1, rand_box_1, img_1, img_2, counts)


# speedup vs baseline: 2202.8958x; 1.2199x over previous
import functools

import jax
import jax.numpy as jnp
from jax import lax
from jax.experimental import pallas as pl
from jax.experimental.pallas import tpu as pltpu
from jax.experimental.pallas import tpu_sc as plsc

_NMS_THR = 0.7
_NUM_INIT = 5000
_PAD = 5120
_MAX_FINAL = 50
_ROUNDS = 49
_NEG = -1.0e30
_ALIVE_THR = -1.0e29
_IMG_HW = 512.0
_MIN_SIDE = _IMG_HW * 0.1

_NTILE = 8
_CHUNK = _PAD // _NTILE
_NCH = _CHUNK // 16


def _sc_nms_body(pkg_hbm, npb_hbm, out_hbm,
                 pkg_v, geo_v, cand_v, all16_v, out_v, npb_v, shared):
    sid = lax.axis_index("s")
    cid = lax.axis_index("c")
    grp = sid // _NTILE
    img = cid * 2 + grp
    g = sid % _NTILE
    base = grp * _NTILE
    obase = _NTILE - base

    pltpu.sync_copy(pkg_hbm.at[img * _NTILE + g], pkg_v)
    pltpu.sync_copy(npb_hbm, npb_v)

    iota16 = lax.iota(jnp.int32, 16)
    npbf = npb_v[...].astype(jnp.float32)
    npb = jnp.max(jnp.where(iota16 == img, npbf, 0.0)).astype(jnp.int32)
    oimg = cid * 2 + (1 - grp)
    npb_oth = jnp.max(jnp.where(iota16 == oimg, npbf, 0.0)).astype(jnp.int32)

    def geo(k, _):
        sl = pl.ds(k * 16, 16)
        r0 = pkg_v[pl.ds(k * 16, 16)]
        r1 = pkg_v[pl.ds(_CHUNK + k * 16, 16)]
        r2 = pkg_v[pl.ds(2 * _CHUNK + k * 16, 16)]
        r3 = pkg_v[pl.ds(3 * _CHUNK + k * 16, 16)]
        s = pkg_v[pl.ds(4 * _CHUNK + k * 16, 16)]
        x1 = jnp.minimum(r0, r2) * _IMG_HW
        x2 = jnp.maximum(r0, r2) * _IMG_HW
        y1 = jnp.minimum(r1, r3) * _IMG_HW
        y2 = jnp.maximum(r1, r3) * _IMG_HW
        w = x2 - x1
        h = y2 - y1
        valid = (h > _MIN_SIDE) & (w > _MIN_SIDE)
        geo_v[pl.ds(k * 16, 16)] = x1
        geo_v[pl.ds(_CHUNK + k * 16, 16)] = y1
        geo_v[pl.ds(2 * _CHUNK + k * 16, 16)] = x2
        geo_v[pl.ds(3 * _CHUNK + k * 16, 16)] = y2
        geo_v[pl.ds(4 * _CHUNK + k * 16, 16)] = w * h
        geo_v[pl.ds(5 * _CHUNK + k * 16, 16)] = jnp.where(valid, s, _NEG)
        return 0

    lax.fori_loop(0, _NCH, geo, 0, unroll=4)

    zeros16 = jnp.zeros((16,), jnp.float32)

    def zr(t, _):
        out_v[pl.ds(t * 16, 16)] = zeros16
        return 0

    lax.fori_loop(0, _MAX_FINAL, zr, 0, unroll=4)

    offs = (jnp.where(iota16 == 2, 0, 0)
            + jnp.where(iota16 == 3, _CHUNK, 0)
            + jnp.where(iota16 == 4, 2 * _CHUNK, 0)
            + jnp.where(iota16 == 5, 3 * _CHUNK, 0)
            + jnp.where(iota16 == 6, 4 * _CHUNK, 0))

    def cond(carry):
        return carry[0]

    def rnd(carry):
        cont, t, bx1, by1, bx2, by2, barea, sup_en, cnt = carry
        sup_b = sup_en > 0.5

        def chunk(k, mc):
            m_v, a_v = mc
            x1c = geo_v[pl.ds(k * 16, 16)]
            y1c = geo_v[pl.ds(_CHUNK + k * 16, 16)]
            x2c = geo_v[pl.ds(2 * _CHUNK + k * 16, 16)]
            y2c = geo_v[pl.ds(3 * _CHUNK + k * 16, 16)]
            ac = geo_v[pl.ds(4 * _CHUNK + k * 16, 16)]
            sc = geo_v[pl.ds(5 * _CHUNK + k * 16, 16)]
            xx1 = jnp.maximum(x1c, bx1)
            yy1 = jnp.maximum(y1c, by1)
            xx2 = jnp.minimum(x2c, bx2)
            yy2 = jnp.minimum(y2c, by2)
            inter = (jnp.maximum(xx2 - xx1, 0.0)
                     * jnp.maximum(yy2 - yy1, 0.0))
            iou = inter / (ac + barea - inter + 1e-9)
            snew = jnp.where((iou > _NMS_THR) & sup_b, _NEG, sc)
            geo_v[pl.ds(5 * _CHUNK + k * 16, 16)] = snew
            take = snew > m_v
            m_v = jnp.where(take, snew, m_v)
            a_v = jnp.where(take, (k * 16 + iota16).astype(jnp.float32), a_v)
            return m_v, a_v

        m_v, a_v = lax.fori_loop(
            0, _NCH, chunk,
            (jnp.full((16,), _NEG, jnp.float32),
             jnp.zeros((16,), jnp.float32)), unroll=4)

        m = jnp.max(m_v)
        am_f = jnp.min(jnp.where(m_v == m, a_v, jnp.float32(1e30)))
        am = jnp.minimum(am_f.astype(jnp.int32), _CHUNK - 1)
        gath = plsc.load_gather(geo_v, [am + offs])
        cand = jnp.where(iota16 == 0, m, gath)
        cand = jnp.where(iota16 == 1,
                         (g * _CHUNK + am).astype(jnp.float32), cand)
        cand_v[...] = cand
        bank = lax.rem(t, 2) * 256
        pltpu.sync_copy(cand_v, shared.at[pl.ds(bank + sid * 16, 16)])
        plsc.subcore_barrier()
        pltpu.sync_copy(shared.at[pl.ds(bank, 256)], all16_v)

        best_m = jnp.float32(_NEG)
        best_i = jnp.float32(1e30)
        nx1 = jnp.float32(0.0)
        ny1 = jnp.float32(0.0)
        nx2 = jnp.float32(0.0)
        ny2 = jnp.float32(0.0)
        nar = jnp.float32(0.0)
        oth_m = jnp.float32(_NEG)
        for j in range(_NTILE):
            row = all16_v[pl.ds(base * 16 + 16 * j, 16)]
            rm = row[0]
            ri = row[1]
            better = (rm > best_m) | ((rm == best_m) & (ri < best_i))
            best_m = jnp.where(better, rm, best_m)
            best_i = jnp.where(better, ri, best_i)
            nx1 = jnp.where(better, row[2], nx1)
            ny1 = jnp.where(better, row[3], ny1)
            nx2 = jnp.where(better, row[4], nx2)
            ny2 = jnp.where(better, row[5], ny2)
            nar = jnp.where(better, row[6], nar)
            orow = all16_v[pl.ds(obase * 16 + 16 * j, 16)]
            oth_m = jnp.maximum(oth_m, orow[0])

        alive = best_m > _ALIVE_THR
        alive_f = jnp.where(alive, 1.0, 0.0).astype(jnp.float32)
        write_f = jnp.where(alive & (t < npb), 1.0, 0.0).astype(jnp.float32)
        vec = jnp.where(iota16 == 0, nx1, 0.0)
        vec = jnp.where(iota16 == 1, ny1, vec)
        vec = jnp.where(iota16 == 2, nx2, vec)
        vec = jnp.where(iota16 == 3, ny2, vec)
        out_v[pl.ds(t * 16, 16)] = vec * write_f
        cnt = cnt + alive.astype(jnp.int32)
        need_own = alive & (t + 1 < npb)
        need_oth = (oth_m > _ALIVE_THR) & (t + 1 < npb_oth)
        cont2 = (need_own | need_oth) & (t + 1 < _ROUNDS)
        return (cont2, t + 1, nx1, ny1, nx2, ny2, nar, alive_f, cnt)

    z = jnp.float32(0.0)
    carry = (jnp.bool_(True), jnp.int32(0), z, z, z, z, z, z, jnp.int32(0))
    carry = lax.while_loop(cond, rnd, carry)
    cnt = carry[8]

    n_final = jnp.minimum(cnt, npb)
    plsc.store_scatter(out_v, [_MAX_FINAL * 16 + iota16],
                       jnp.full((16,), n_final.astype(jnp.float32)))

    @pl.when(g == 0)
    def _():
        pltpu.sync_copy(out_v, out_hbm.at[img])


def _split_body(a_ref, b_ref, o1_ref, o2_ref):
    o1_ref[...] = a_ref[...]
    o2_ref[...] = b_ref[...]


@functools.partial(jax.jit, static_argnames=())
def kernel(img, rand_boxes_init, pseudo_scores, num_boxes_per_img):
    nimg = img.shape[0]
    n = rand_boxes_init.shape[1]

    rt = jnp.moveaxis(rand_boxes_init, 2, 1)
    rt = jnp.pad(rt, ((0, 0), (0, 0), (0, _PAD - n)))
    sc = jnp.pad(pseudo_scores, ((0, 0), (0, _PAD - n)))
    pkg = jnp.concatenate([rt, sc[:, None, :]], axis=1)
    pkg = pkg.reshape(nimg, 5, _NTILE, _CHUNK)
    pkg = jnp.moveaxis(pkg, 1, 2).reshape(nimg * _NTILE, 5 * _CHUNK)
    npb16 = jnp.pad(num_boxes_per_img, (0, 16 - nimg))

    mesh = plsc.VectorSubcoreMesh(core_axis_name="c", subcore_axis_name="s")
    out_flat = pl.kernel(
        _sc_nms_body,
        out_type=jax.ShapeDtypeStruct((nimg, 1024), jnp.float32),
        mesh=mesh,
        compiler_params=pltpu.CompilerParams(needs_layout_passes=False),
        scratch_types=[
            pltpu.VMEM((5 * _CHUNK,), jnp.float32),
            pltpu.VMEM((6 * _CHUNK,), jnp.float32),
            pltpu.VMEM((16,), jnp.float32),
            pltpu.VMEM((256,), jnp.float32),
            pltpu.VMEM((1024,), jnp.float32),
            pltpu.VMEM((16,), jnp.int32),
            pltpu.VMEM_SHARED((512,), jnp.float32),
        ],
    )(pkg, npb16)

    rand_box_1 = out_flat[:, :_MAX_FINAL * 16].reshape(
        nimg, _MAX_FINAL, 16)[:, :, :4]
    counts = out_flat[:, _MAX_FINAL * 16].astype(jnp.int32)

    ch = img.shape[1] // 2
    hw = img.shape[2:]
    img_1, img_2 = pl.pallas_call(
        _split_body,
        grid=(nimg, ch),
        in_specs=[
            pl.BlockSpec((1, 1) + hw, lambda i, c: (i, c, 0, 0)),
            pl.BlockSpec((1, 1) + hw, lambda i, c, _ch=ch: (i, c + _ch, 0, 0)),
        ],
        out_specs=[
            pl.BlockSpec((1, 1) + hw, lambda i, c: (i, c, 0, 0)),
            pl.BlockSpec((1, 1) + hw, lambda i, c: (i, c, 0, 0)),
        ],
        out_shape=[
            jax.ShapeDtypeStruct((nimg, ch) + hw, img.dtype),
            jax.ShapeDtypeStruct((nimg, ch) + hw, img.dtype),
        ],
    )(img, img)

    return (rand_box_1, rand_box_1, img_1, img_2, counts)


# speedup vs baseline: 2224.0770x; 1.0096x over previous
import functools

import jax
import jax.numpy as jnp
from jax import lax
from jax.experimental import pallas as pl
from jax.experimental.pallas import tpu as pltpu
from jax.experimental.pallas import tpu_sc as plsc

_NMS_THR = 0.7
_NUM_INIT = 5000
_PAD = 5120
_MAX_FINAL = 50
_ROUNDS = 49
_NEG = -1.0e30
_ALIVE_THR = -1.0e29
_IMG_HW = 512.0
_MIN_SIDE = _IMG_HW * 0.1

_NTILE = 8
_CHUNK = _PAD // _NTILE
_NCH = _CHUNK // 16


def _sc_nms_body(pkg_hbm, npb_hbm, out_hbm,
                 pkg_v, geo_v, cand_v, all16_v, out_v, npb_v, shared):
    sid = lax.axis_index("s")
    cid = lax.axis_index("c")
    grp = sid // _NTILE
    img = cid * 2 + grp
    g = sid % _NTILE
    base = grp * _NTILE
    obase = _NTILE - base

    pltpu.sync_copy(pkg_hbm.at[img * _NTILE + g], pkg_v)
    pltpu.sync_copy(npb_hbm, npb_v)

    iota16 = lax.iota(jnp.int32, 16)
    npbf = npb_v[...].astype(jnp.float32)
    npb = jnp.max(jnp.where(iota16 == img, npbf, 0.0)).astype(jnp.int32)
    oimg = cid * 2 + (1 - grp)
    npb_oth = jnp.max(jnp.where(iota16 == oimg, npbf, 0.0)).astype(jnp.int32)

    def geo(k, _):
        sl = pl.ds(k * 16, 16)
        r0 = pkg_v[pl.ds(k * 16, 16)]
        r1 = pkg_v[pl.ds(_CHUNK + k * 16, 16)]
        r2 = pkg_v[pl.ds(2 * _CHUNK + k * 16, 16)]
        r3 = pkg_v[pl.ds(3 * _CHUNK + k * 16, 16)]
        s = pkg_v[pl.ds(4 * _CHUNK + k * 16, 16)]
        x1 = jnp.minimum(r0, r2) * _IMG_HW
        x2 = jnp.maximum(r0, r2) * _IMG_HW
        y1 = jnp.minimum(r1, r3) * _IMG_HW
        y2 = jnp.maximum(r1, r3) * _IMG_HW
        w = x2 - x1
        h = y2 - y1
        valid = (h > _MIN_SIDE) & (w > _MIN_SIDE)
        geo_v[pl.ds(k * 16, 16)] = x1
        geo_v[pl.ds(_CHUNK + k * 16, 16)] = y1
        geo_v[pl.ds(2 * _CHUNK + k * 16, 16)] = x2
        geo_v[pl.ds(3 * _CHUNK + k * 16, 16)] = y2
        geo_v[pl.ds(4 * _CHUNK + k * 16, 16)] = w * h
        geo_v[pl.ds(5 * _CHUNK + k * 16, 16)] = jnp.where(valid, s, _NEG)
        return 0

    lax.fori_loop(0, _NCH, geo, 0, unroll=4)

    zeros16 = jnp.zeros((16,), jnp.float32)

    def zr(t, _):
        out_v[pl.ds(t * 16, 16)] = zeros16
        return 0

    lax.fori_loop(0, _MAX_FINAL, zr, 0, unroll=4)

    offs = (jnp.where(iota16 == 2, 0, 0)
            + jnp.where(iota16 == 3, _CHUNK, 0)
            + jnp.where(iota16 == 4, 2 * _CHUNK, 0)
            + jnp.where(iota16 == 5, 3 * _CHUNK, 0)
            + jnp.where(iota16 == 6, 4 * _CHUNK, 0))

    def cond(carry):
        return carry[0]

    def rnd(carry):
        cont, t, bx1, by1, bx2, by2, barea, sup_en, cnt = carry
        sup_b = sup_en > 0.5

        def chunk(k, mc):
            m_v, a_v = mc
            x1c = geo_v[pl.ds(k * 16, 16)]
            y1c = geo_v[pl.ds(_CHUNK + k * 16, 16)]
            x2c = geo_v[pl.ds(2 * _CHUNK + k * 16, 16)]
            y2c = geo_v[pl.ds(3 * _CHUNK + k * 16, 16)]
            ac = geo_v[pl.ds(4 * _CHUNK + k * 16, 16)]
            sc = geo_v[pl.ds(5 * _CHUNK + k * 16, 16)]
            xx1 = jnp.maximum(x1c, bx1)
            yy1 = jnp.maximum(y1c, by1)
            xx2 = jnp.minimum(x2c, bx2)
            yy2 = jnp.minimum(y2c, by2)
            inter = (jnp.maximum(xx2 - xx1, 0.0)
                     * jnp.maximum(yy2 - yy1, 0.0))
            iou = inter / (ac + barea - inter + 1e-9)
            snew = jnp.where((iou > _NMS_THR) & sup_b, _NEG, sc)
            geo_v[pl.ds(5 * _CHUNK + k * 16, 16)] = snew
            take = snew > m_v
            m_v = jnp.where(take, snew, m_v)
            a_v = jnp.where(take, (k * 16 + iota16).astype(jnp.float32), a_v)
            return m_v, a_v

        m_v, a_v = lax.fori_loop(
            0, _NCH, chunk,
            (jnp.full((16,), _NEG, jnp.float32),
             jnp.zeros((16,), jnp.float32)), unroll=4)

        m = jnp.max(m_v)
        am_f = jnp.min(jnp.where(m_v == m, a_v, jnp.float32(1e30)))
        am = jnp.minimum(am_f.astype(jnp.int32), _CHUNK - 1)
        gath = plsc.load_gather(geo_v, [am + offs])
        cand = jnp.where(iota16 == 0, m, gath)
        cand = jnp.where(iota16 == 1,
                         (g * _CHUNK + am).astype(jnp.float32), cand)
        cand_v[...] = cand
        bank = lax.rem(t, 2) * 256
        pltpu.sync_copy(cand_v, shared.at[pl.ds(bank + sid * 16, 16)])
        plsc.subcore_barrier()
        pltpu.sync_copy(shared.at[pl.ds(bank, 256)], all16_v)

        mv = plsc.load_gather(all16_v, [iota16 * 16])
        iv = plsc.load_gather(all16_v, [iota16 * 16 + 1])
        own = (iota16 >= base) & (iota16 < base + _NTILE)
        mown = jnp.where(own, mv, _NEG)
        oth_m = jnp.max(jnp.where(own, _NEG, mv))
        best_m = jnp.max(mown)
        best_i = jnp.min(jnp.where(mown == best_m, iv, jnp.float32(1e30)))
        lane_f = iota16.astype(jnp.float32)
        j_f = jnp.min(jnp.where((mown == best_m) & (iv == best_i),
                                lane_f, jnp.float32(1e30)))
        jrow = jnp.minimum(j_f.astype(jnp.int32), 15)
        row = plsc.load_gather(all16_v, [jrow * 16 + iota16])
        nx1 = row[2]
        ny1 = row[3]
        nx2 = row[4]
        ny2 = row[5]
        nar = row[6]

        alive = best_m > _ALIVE_THR
        alive_f = jnp.where(alive, 1.0, 0.0).astype(jnp.float32)
        write_f = jnp.where(alive & (t < npb), 1.0, 0.0).astype(jnp.float32)
        vec = jnp.where(iota16 == 0, nx1, 0.0)
        vec = jnp.where(iota16 == 1, ny1, vec)
        vec = jnp.where(iota16 == 2, nx2, vec)
        vec = jnp.where(iota16 == 3, ny2, vec)
        out_v[pl.ds(t * 16, 16)] = vec * write_f
        cnt = cnt + alive.astype(jnp.int32)
        need_own = alive & (t + 1 < npb)
        need_oth = (oth_m > _ALIVE_THR) & (t + 1 < npb_oth)
        cont2 = (need_own | need_oth) & (t + 1 < _ROUNDS)
        return (cont2, t + 1, nx1, ny1, nx2, ny2, nar, alive_f, cnt)

    z = jnp.float32(0.0)
    carry = (jnp.bool_(True), jnp.int32(0), z, z, z, z, z, z, jnp.int32(0))
    carry = lax.while_loop(cond, rnd, carry)
    cnt = carry[8]

    n_final = jnp.minimum(cnt, npb)
    plsc.store_scatter(out_v, [_MAX_FINAL * 16 + iota16],
                       jnp.full((16,), n_final.astype(jnp.float32)))

    @pl.when(g == 0)
    def _():
        pltpu.sync_copy(out_v, out_hbm.at[img])


def _split_body(a_ref, b_ref, o1_ref, o2_ref):
    o1_ref[...] = a_ref[...]
    o2_ref[...] = b_ref[...]


@functools.partial(jax.jit, static_argnames=())
def kernel(img, rand_boxes_init, pseudo_scores, num_boxes_per_img):
    nimg = img.shape[0]
    n = rand_boxes_init.shape[1]

    ch = img.shape[1] // 2
    hw = img.shape[2:]
    img_1, img_2 = pl.pallas_call(
        _split_body,
        grid=(nimg, ch),
        in_specs=[
            pl.BlockSpec((1, 1) + hw, lambda i, c: (i, c, 0, 0)),
            pl.BlockSpec((1, 1) + hw, lambda i, c, _ch=ch: (i, c + _ch, 0, 0)),
        ],
        out_specs=[
            pl.BlockSpec((1, 1) + hw, lambda i, c: (i, c, 0, 0)),
            pl.BlockSpec((1, 1) + hw, lambda i, c: (i, c, 0, 0)),
        ],
        out_shape=[
            jax.ShapeDtypeStruct((nimg, ch) + hw, img.dtype),
            jax.ShapeDtypeStruct((nimg, ch) + hw, img.dtype),
        ],
    )(img, img)

    rt = jnp.moveaxis(rand_boxes_init, 2, 1)
    rt = jnp.pad(rt, ((0, 0), (0, 0), (0, _PAD - n)))
    sc = jnp.pad(pseudo_scores, ((0, 0), (0, _PAD - n)))
    pkg = jnp.concatenate([rt, sc[:, None, :]], axis=1)
    pkg = pkg.reshape(nimg, 5, _NTILE, _CHUNK)
    pkg = jnp.moveaxis(pkg, 1, 2).reshape(nimg * _NTILE, 5 * _CHUNK)
    npb16 = jnp.pad(num_boxes_per_img, (0, 16 - nimg))

    mesh = plsc.VectorSubcoreMesh(core_axis_name="c", subcore_axis_name="s")
    out_flat = pl.kernel(
        _sc_nms_body,
        out_type=jax.ShapeDtypeStruct((nimg, 1024), jnp.float32),
        mesh=mesh,
        compiler_params=pltpu.CompilerParams(needs_layout_passes=False),
        scratch_types=[
            pltpu.VMEM((5 * _CHUNK,), jnp.float32),
            pltpu.VMEM((6 * _CHUNK,), jnp.float32),
            pltpu.VMEM((16,), jnp.float32),
            pltpu.VMEM((256,), jnp.float32),
            pltpu.VMEM((1024,), jnp.float32),
            pltpu.VMEM((16,), jnp.int32),
            pltpu.VMEM_SHARED((512,), jnp.float32),
        ],
    )(pkg, npb16)

    rand_box_1 = out_flat[:, :_MAX_FINAL * 16].reshape(
        nimg, _MAX_FINAL, 16)[:, :, :4]
    counts = out_flat[:, _MAX_FINAL * 16].astype(jnp.int32)

    return (rand_box_1, rand_box_1, img_1, img_2, counts)


# speedup vs baseline: 2739.1879x; 1.2316x over previous
import functools

import jax
import jax.numpy as jnp
from jax import lax
from jax.experimental import pallas as pl
from jax.experimental.pallas import tpu as pltpu
from jax.experimental.pallas import tpu_sc as plsc

_NMS_THR = 0.7
_NUM_INIT = 5000
_PAD = 5120
_MAX_FINAL = 50
_ROUNDS = 49
_NEG = -1.0e30
_ALIVE_THR = -1.0e29
_IMG_HW = 512.0
_MIN_SIDE = _IMG_HW * 0.1

_NTILE = 8
_CHUNK = _PAD // _NTILE
_NCH = _CHUNK // 16


def _sc_nms_body(pkg_hbm, npb_hbm, out_hbm,
                 pkg_v, geo_v, cand_v, all16_v, out_v, npb_v, shared):
    sid = lax.axis_index("s")
    cid = lax.axis_index("c")
    grp = sid // _NTILE
    img = cid * 2 + grp
    g = sid % _NTILE
    base = grp * _NTILE
    obase = _NTILE - base

    pltpu.sync_copy(pkg_hbm.at[img * _NTILE + g], pkg_v)
    pltpu.sync_copy(npb_hbm, npb_v)

    iota16 = lax.iota(jnp.int32, 16)
    npbf = npb_v[...].astype(jnp.float32)
    npb = jnp.max(jnp.where(iota16 == img, npbf, 0.0)).astype(jnp.int32)
    oimg = cid * 2 + (1 - grp)
    npb_oth = jnp.max(jnp.where(iota16 == oimg, npbf, 0.0)).astype(jnp.int32)

    def geo(k, _):
        sl = pl.ds(k * 16, 16)
        r0 = pkg_v[pl.ds(k * 16, 16)]
        r1 = pkg_v[pl.ds(_CHUNK + k * 16, 16)]
        r2 = pkg_v[pl.ds(2 * _CHUNK + k * 16, 16)]
        r3 = pkg_v[pl.ds(3 * _CHUNK + k * 16, 16)]
        s = pkg_v[pl.ds(4 * _CHUNK + k * 16, 16)]
        x1 = jnp.minimum(r0, r2) * _IMG_HW
        x2 = jnp.maximum(r0, r2) * _IMG_HW
        y1 = jnp.minimum(r1, r3) * _IMG_HW
        y2 = jnp.maximum(r1, r3) * _IMG_HW
        w = x2 - x1
        h = y2 - y1
        valid = (h > _MIN_SIDE) & (w > _MIN_SIDE)
        geo_v[pl.ds(k * 16, 16)] = x1
        geo_v[pl.ds(_CHUNK + k * 16, 16)] = y1
        geo_v[pl.ds(2 * _CHUNK + k * 16, 16)] = x2
        geo_v[pl.ds(3 * _CHUNK + k * 16, 16)] = y2
        geo_v[pl.ds(4 * _CHUNK + k * 16, 16)] = w * h
        geo_v[pl.ds(5 * _CHUNK + k * 16, 16)] = jnp.where(valid, s, _NEG)
        return 0

    lax.fori_loop(0, _NCH, geo, 0, unroll=4)

    zeros16 = jnp.zeros((16,), jnp.float32)

    def zr(t, _):
        out_v[pl.ds(t * 16, 16)] = zeros16
        return 0

    lax.fori_loop(0, _MAX_FINAL, zr, 0, unroll=4)

    offs = (jnp.where(iota16 == 2, 0, 0)
            + jnp.where(iota16 == 3, _CHUNK, 0)
            + jnp.where(iota16 == 4, 2 * _CHUNK, 0)
            + jnp.where(iota16 == 5, 3 * _CHUNK, 0)
            + jnp.where(iota16 == 6, 4 * _CHUNK, 0))

    def cond(carry):
        return carry[0]

    def rnd(carry):
        cont, t, bx1, by1, bx2, by2, barea, sup_en, cnt = carry
        sup_b = sup_en > 0.5

        def chunk(k, mc):
            m_v, a_v = mc
            x1c = geo_v[pl.ds(k * 16, 16)]
            y1c = geo_v[pl.ds(_CHUNK + k * 16, 16)]
            x2c = geo_v[pl.ds(2 * _CHUNK + k * 16, 16)]
            y2c = geo_v[pl.ds(3 * _CHUNK + k * 16, 16)]
            ac = geo_v[pl.ds(4 * _CHUNK + k * 16, 16)]
            sc = geo_v[pl.ds(5 * _CHUNK + k * 16, 16)]
            xx1 = jnp.maximum(x1c, bx1)
            yy1 = jnp.maximum(y1c, by1)
            xx2 = jnp.minimum(x2c, bx2)
            yy2 = jnp.minimum(y2c, by2)
            inter = (jnp.maximum(xx2 - xx1, 0.0)
                     * jnp.maximum(yy2 - yy1, 0.0))
            iou = inter / (ac + barea - inter + 1e-9)
            snew = jnp.where((iou > _NMS_THR) & sup_b, _NEG, sc)
            geo_v[pl.ds(5 * _CHUNK + k * 16, 16)] = snew
            take = snew > m_v
            m_v = jnp.where(take, snew, m_v)
            a_v = jnp.where(take, (k * 16 + iota16).astype(jnp.float32), a_v)
            return m_v, a_v

        m_v, a_v = lax.fori_loop(
            0, _NCH, chunk,
            (jnp.full((16,), _NEG, jnp.float32),
             jnp.zeros((16,), jnp.float32)), unroll=8)

        m = jnp.max(m_v)
        am_f = jnp.min(jnp.where(m_v == m, a_v, jnp.float32(1e30)))
        am = jnp.minimum(am_f.astype(jnp.int32), _CHUNK - 1)
        gath = plsc.load_gather(geo_v, [am + offs])
        cand = jnp.where(iota16 == 0, m, gath)
        cand = jnp.where(iota16 == 1,
                         (g * _CHUNK + am).astype(jnp.float32), cand)
        cand_v[...] = cand
        bank = lax.rem(t, 2) * 256
        pltpu.sync_copy(cand_v, shared.at[pl.ds(bank + sid * 16, 16)])
        plsc.subcore_barrier()
        pltpu.sync_copy(shared.at[pl.ds(bank, 256)], all16_v)

        mv = plsc.load_gather(all16_v, [iota16 * 16])
        iv = plsc.load_gather(all16_v, [iota16 * 16 + 1])
        own = (iota16 >= base) & (iota16 < base + _NTILE)
        mown = jnp.where(own, mv, _NEG)
        oth_m = jnp.max(jnp.where(own, _NEG, mv))
        best_m = jnp.max(mown)
        best_i = jnp.min(jnp.where(mown == best_m, iv, jnp.float32(1e30)))
        lane_f = iota16.astype(jnp.float32)
        j_f = jnp.min(jnp.where((mown == best_m) & (iv == best_i),
                                lane_f, jnp.float32(1e30)))
        jrow = jnp.minimum(j_f.astype(jnp.int32), 15)
        row = plsc.load_gather(all16_v, [jrow * 16 + iota16])
        nx1 = row[2]
        ny1 = row[3]
        nx2 = row[4]
        ny2 = row[5]
        nar = row[6]

        alive = best_m > _ALIVE_THR
        alive_f = jnp.where(alive, 1.0, 0.0).astype(jnp.float32)
        write_f = jnp.where(alive & (t < npb), 1.0, 0.0).astype(jnp.float32)
        vec = jnp.where(iota16 == 0, nx1, 0.0)
        vec = jnp.where(iota16 == 1, ny1, vec)
        vec = jnp.where(iota16 == 2, nx2, vec)
        vec = jnp.where(iota16 == 3, ny2, vec)
        out_v[pl.ds(t * 16, 16)] = vec * write_f
        cnt = cnt + alive.astype(jnp.int32)
        need_own = alive & (t + 1 < npb)
        need_oth = (oth_m > _ALIVE_THR) & (t + 1 < npb_oth)
        cont2 = (need_own | need_oth) & (t + 1 < _ROUNDS)
        return (cont2, t + 1, nx1, ny1, nx2, ny2, nar, alive_f, cnt)

    z = jnp.float32(0.0)
    carry = (jnp.bool_(True), jnp.int32(0), z, z, z, z, z, z, jnp.int32(0))
    carry = lax.while_loop(cond, rnd, carry)
    cnt = carry[8]

    n_final = jnp.minimum(cnt, npb)
    plsc.store_scatter(out_v, [_MAX_FINAL * 16 + iota16],
                       jnp.full((16,), n_final.astype(jnp.float32)))

    @pl.when(g == 0)
    def _():
        pltpu.sync_copy(out_v, out_hbm.at[img])


def _split_body(a_ref, b_ref, o1_ref, o2_ref):
    o1_ref[...] = a_ref[...]
    o2_ref[...] = b_ref[...]


@functools.partial(jax.jit, static_argnames=())
def kernel(img, rand_boxes_init, pseudo_scores, num_boxes_per_img):
    nimg = img.shape[0]
    n = rand_boxes_init.shape[1]

    ch = img.shape[1] // 2
    hw = img.shape[2:]
    img_1, img_2 = pl.pallas_call(
        _split_body,
        grid=(nimg, ch),
        in_specs=[
            pl.BlockSpec((1, 1) + hw, lambda i, c: (i, c, 0, 0)),
            pl.BlockSpec((1, 1) + hw, lambda i, c, _ch=ch: (i, c + _ch, 0, 0)),
        ],
        out_specs=[
            pl.BlockSpec((1, 1) + hw, lambda i, c: (i, c, 0, 0)),
            pl.BlockSpec((1, 1) + hw, lambda i, c: (i, c, 0, 0)),
        ],
        out_shape=[
            jax.ShapeDtypeStruct((nimg, ch) + hw, img.dtype),
            jax.ShapeDtypeStruct((nimg, ch) + hw, img.dtype),
        ],
    )(img, img)

    rt = jnp.moveaxis(rand_boxes_init, 2, 1)
    rt = jnp.pad(rt, ((0, 0), (0, 0), (0, _PAD - n)))
    sc = jnp.pad(pseudo_scores, ((0, 0), (0, _PAD - n)))
    pkg = jnp.concatenate([rt, sc[:, None, :]], axis=1)
    pkg = pkg.reshape(nimg, 5, _NTILE, _CHUNK)
    pkg = jnp.moveaxis(pkg, 1, 2).reshape(nimg * _NTILE, 5 * _CHUNK)
    npb16 = jnp.pad(num_boxes_per_img, (0, 16 - nimg))

    mesh = plsc.VectorSubcoreMesh(core_axis_name="c", subcore_axis_name="s")
    out_flat = pl.kernel(
        _sc_nms_body,
        out_type=jax.ShapeDtypeStruct((nimg, 1024), jnp.float32),
        mesh=mesh,
        compiler_params=pltpu.CompilerParams(needs_layout_passes=False),
        scratch_types=[
            pltpu.VMEM((5 * _CHUNK,), jnp.float32),
            pltpu.VMEM((6 * _CHUNK,), jnp.float32),
            pltpu.VMEM((16,), jnp.float32),
            pltpu.VMEM((256,), jnp.float32),
            pltpu.VMEM((1024,), jnp.float32),
            pltpu.VMEM((16,), jnp.int32),
            pltpu.VMEM_SHARED((512,), jnp.float32),
        ],
    )(pkg, npb16)

    rand_box_1 = out_flat[:, :_MAX_FINAL * 16].reshape(
        nimg, _MAX_FINAL, 16)[:, :, :4]
    counts = out_flat[:, _MAX_FINAL * 16].astype(jnp.int32)

    return (rand_box_1, rand_box_1, img_1, img_2, counts)
